# baseline (speedup 1.0000x reference)
import jax
import jax.numpy as jnp
from jax.experimental import pallas as pl
from jax.experimental.pallas import tpu as pltpu

B = 64
K = 64 * 64 * 64
OC = 2
KC = 8192


def _body(x_ref, m_ref, w_ref, b_ref, out_ref):
    k = pl.program_id(0)
    xv = x_ref[...]
    gated = jnp.where(jnp.abs(xv) > 1e-8, xv, 0.0) * m_ref[...]
    part = jax.lax.dot_general(
        gated, w_ref[...], (((1,), (1,)), ((), ())),
        preferred_element_type=jnp.float32)

    @pl.when(k == 0)
    def _init():
        out_ref[...] = part + b_ref[...]

    @pl.when(k != 0)
    def _acc():
        out_ref[...] += part


def kernel(x, mask, weight, bias):
    x2 = x.reshape(B, K)
    m2 = mask.reshape(1, K)
    w2 = weight.reshape(OC, K)
    b2 = bias.reshape(1, OC)
    out = pl.pallas_call(
        _body,
        grid=(K // KC,),
        in_specs=[
            pl.BlockSpec((B, KC), lambda k: (0, k)),
            pl.BlockSpec((1, KC), lambda k: (0, k)),
            pl.BlockSpec((OC, KC), lambda k: (0, k)),
            pl.BlockSpec((1, OC), lambda k: (0, 0)),
        ],
        out_specs=pl.BlockSpec((B, OC), lambda k: (0, 0)),
        out_shape=jax.ShapeDtypeStruct((B, OC), jnp.float32),
        compiler_params=pltpu.CompilerParams(
            dimension_semantics=("arbitrary",)),
    )(x2, m2, w2, b2)
    return out
